# Initial kernel scaffold; baseline (speedup 1.0000x reference)
import jax
import jax.numpy as jnp
from jax.experimental import pallas as pl


def kernel(x, edge_index, batch, W1, b1, W2, b2, W3, b3, W4, b4, W5, b5, LW1, LB1, LW2, LB2, LW3, LB3):
    raise NotImplementedError("write your pallas kernel here")


# speedup vs baseline: 7.1970x; 7.1970x over previous
import jax
import jax.numpy as jnp
from jax import lax
from jax.experimental import pallas as pl
from jax.experimental.pallas import tpu as pltpu
from jax.experimental.pallas import tpu_sc as plsc

_N = 50000
_E = 800000
_G = 64
_DIN = 336
_DH = 100
_DP = 112
_DOUT = 29
_NG = 7
_L = 16
_ER = 6400
_RW = 125
_TROWS = _ER // 16
_CH = 8
_NCH = _TROWS // _CH
_ZR = _N // 16
_ZC = _ZR // _RW
_OWNER = (0, 0, 0, 0, 1, 1, 1)
_BN = 1000
_GRID = _N // _BN
_DEG_TR = _ER // 32
_DEG_NCH = _DEG_TR // _CH
_BR = 400
_WCH = 1000
_NWC = _N // _WCH


def _acc_chunks(body):
    sub = lax.axis_index("s")

    def _k(k, c):
        ch = sub + 16 * k

        @pl.when(ch < _NWC)
        def _():
            body(ch)
        return c
    lax.fori_loop(0, (_NWC + 15) // 16, _k, 0)


def _sc_mesh():
    return plsc.VectorSubcoreMesh(core_axis_name="c", subcore_axis_name="s")


def _deg_body(dst_hbm, degp, didx, ones, zbuf, obuf, acc, ssem):
    core = lax.axis_index("c")
    sub = lax.axis_index("s")

    def _init(i, c):
        ones[i, :] = jnp.ones((_L,), jnp.float32)
        return c
    lax.fori_loop(0, _RW, _init, 0)

    def _initz(i, c):
        zbuf[i, :] = jnp.zeros((_L,), jnp.float32)
        return c
    lax.fori_loop(0, _WCH, _initz, 0)

    _acc_chunks(lambda ch: pltpu.sync_copy(zbuf, acc.at[pl.ds(ch * _WCH, _WCH)]))
    plsc.subcore_barrier()

    def _chunk(cidx, c):
        base = core * (_ER // 2) + sub * _DEG_TR + cidx * _CH
        pltpu.sync_copy(dst_hbm.at[pl.ds(base, _CH)], didx)
        cps = [pltpu.async_copy(ones, acc.at[didx.at[j]], ssem, add=True)
               for j in range(_CH)]
        for cp in cps:
            cp.wait()
        return c
    lax.fori_loop(0, _DEG_NCH, _chunk, 0)
    plsc.subcore_barrier()

    def _wout(ch):
        pltpu.sync_copy(acc.at[pl.ds(ch * _WCH, _WCH)], obuf)
        pltpu.sync_copy(obuf, degp.at[core, pl.ds(ch * _WCH, _WCH)])
    _acc_chunks(_wout)


_deg = pl.kernel(
    _deg_body,
    out_type=jax.ShapeDtypeStruct((2, _N, _L), jnp.float32),
    mesh=_sc_mesh(),
    compiler_params=pltpu.CompilerParams(use_tc_tiling_on_sc=False),
    scratch_types=[
        pltpu.VMEM((_CH, _RW), jnp.int32),
        pltpu.VMEM((_RW, _L), jnp.float32),
        pltpu.VMEM((_WCH, _L), jnp.float32),
        pltpu.VMEM((_WCH, _L), jnp.float32),
        pltpu.VMEM_SHARED((_N, _L), jnp.float32),
        pltpu.SemaphoreType.DMA,
    ],
)


def _edge_agg_body(src_hbm, dst_hbm, *refs):
    u = refs[0:_NG]
    agg = refs[_NG:2 * _NG]
    sidx, didx, rows, zbuf, obuf, acc, gsem, ssem = refs[2 * _NG:]
    core = lax.axis_index("c")
    sub = lax.axis_index("s")

    def _zb(i, c):
        zbuf[i, :] = jnp.zeros((_L,), jnp.float32)
        return c
    lax.fori_loop(0, _WCH, _zb, 0)

    for g in range(_NG):
        @pl.when(core == _OWNER[g])
        def _(g=g):
            _acc_chunks(
                lambda ch: pltpu.sync_copy(zbuf, acc.at[pl.ds(ch * _WCH, _WCH)]))
            plsc.subcore_barrier()

            def _chunk(cidx, c):
                base = sub * _TROWS + cidx * _CH
                pltpu.sync_copy(src_hbm.at[pl.ds(base, _CH)], sidx)
                pltpu.sync_copy(dst_hbm.at[pl.ds(base, _CH)], didx)
                gcps = [pltpu.async_copy(u[g].at[sidx.at[j]], rows.at[j], gsem)
                        for j in range(_CH)]
                for cp in gcps:
                    cp.wait()
                scps = [pltpu.async_copy(rows.at[j], acc.at[didx.at[j]], ssem,
                                         add=True)
                        for j in range(_CH)]
                for cp in scps:
                    cp.wait()
                return c
            lax.fori_loop(0, _NCH, _chunk, 0)
            plsc.subcore_barrier()

            def _wout(ch):
                pltpu.sync_copy(acc.at[pl.ds(ch * _WCH, _WCH)], obuf)
                pltpu.sync_copy(obuf, agg[g].at[pl.ds(ch * _WCH, _WCH)])
            _acc_chunks(_wout)
            plsc.subcore_barrier()


_edge_agg = pl.kernel(
    _edge_agg_body,
    out_type=[jax.ShapeDtypeStruct((_N, _L), jnp.float32)] * _NG,
    mesh=_sc_mesh(),
    compiler_params=pltpu.CompilerParams(use_tc_tiling_on_sc=False),
    scratch_types=[
        pltpu.VMEM((_CH, _RW), jnp.int32),
        pltpu.VMEM((_CH, _RW), jnp.int32),
        pltpu.VMEM((_CH, _RW, _L), jnp.float32),
        pltpu.VMEM((_WCH, _L), jnp.float32),
        pltpu.VMEM((_WCH, _L), jnp.float32),
        pltpu.VMEM_SHARED((_N, _L), jnp.float32),
        pltpu.SemaphoreType.DMA,
        pltpu.SemaphoreType.DMA,
    ],
)


def _pool_body(h5_hbm, b_hbm, sums_p, counts_p,
               bidx, hbuf, ones, zbuf, zcbuf, accs, accc, ssem):
    core = lax.axis_index("c")
    sub = lax.axis_index("s")
    w = core * 16 + sub

    def _init(i, c):
        ones[i, :] = jnp.ones((_L,), jnp.float32)
        return c
    lax.fori_loop(0, _RW, _init, 0)

    @pl.when(sub == 0)
    def _():
        def _z(i, c):
            for j in range(_DP // _L):
                zbuf[i, pl.ds(j * _L, _L)] = jnp.zeros((_L,), jnp.float32)
            zcbuf[i, :] = jnp.zeros((_L,), jnp.float32)
            return c
        lax.fori_loop(0, _G, _z, 0)
        pltpu.sync_copy(zbuf, accs)
        pltpu.sync_copy(zcbuf, accc)
    plsc.subcore_barrier()

    def _do_chunk(ch):
        pltpu.sync_copy(b_hbm.at[pl.ds(ch * _CH, _CH)], bidx)
        for j in range(_CH):
            row = ch * _CH + j
            pltpu.sync_copy(h5_hbm.at[row], hbuf)
            pltpu.sync_copy(hbuf, accs.at[bidx.at[j]], add=True)
            pltpu.sync_copy(ones, accc.at[bidx.at[j]], add=True)

    first = jnp.where(w < 18, 2 * w, w + 18)
    _do_chunk(first)

    @pl.when(w < 18)
    def _():
        _do_chunk(2 * w + 1)
    plsc.subcore_barrier()

    @pl.when(sub == 0)
    def _():
        pltpu.sync_copy(accs, zbuf)
        pltpu.sync_copy(zbuf, sums_p.at[core])
        pltpu.sync_copy(accc, zcbuf)
        pltpu.sync_copy(zcbuf, counts_p.at[core])


_pool = pl.kernel(
    _pool_body,
    out_type=[jax.ShapeDtypeStruct((2, _G, _DP), jnp.float32),
              jax.ShapeDtypeStruct((2, _G, _L), jnp.float32)],
    mesh=_sc_mesh(),
    compiler_params=pltpu.CompilerParams(use_tc_tiling_on_sc=False),
    scratch_types=[
        pltpu.VMEM((_CH, _RW), jnp.int32),
        pltpu.VMEM((_RW, _DP), jnp.float32),
        pltpu.VMEM((_RW, _L), jnp.float32),
        pltpu.VMEM((_G, _DP), jnp.float32),
        pltpu.VMEM((_G, _L), jnp.float32),
        pltpu.VMEM_SHARED((_G, _DP), jnp.float32),
        pltpu.VMEM_SHARED((_G, _L), jnp.float32),
        pltpu.SemaphoreType.DMA,
    ],
)


def _mm1_body(x_ref, w_ref, degp_ref, *outs):
    deg = degp_ref[0][:, 0:1] + degp_ref[1][:, 0:1] + 2.0
    dis = lax.rsqrt(deg)
    t = jnp.dot(x_ref[...], w_ref[...], preferred_element_type=jnp.float32)
    un = dis * t
    for g in range(_NG):
        outs[g][...] = un[:, g * _L:(g + 1) * _L]


_k1 = pl.pallas_call(
    _mm1_body,
    grid=(_GRID,),
    in_specs=[
        pl.BlockSpec((_BN, _DIN), lambda i: (i, 0)),
        pl.BlockSpec((_DIN, _DP), lambda i: (0, 0)),
        pl.BlockSpec((2, _BN, _L), lambda i: (0, i, 0)),
    ],
    out_specs=[pl.BlockSpec((_BN, _L), lambda i: (i, 0))] * _NG,
    out_shape=[jax.ShapeDtypeStruct((_N, _L), jnp.float32)] * _NG,
)


def _gcn_mid_body(degp_ref, w_ref, b_ref, *refs):
    aggs = refs[0:_NG]
    us = refs[_NG:2 * _NG]
    outs = refs[2 * _NG:3 * _NG]
    hbuf = refs[3 * _NG]
    deg = degp_ref[0][:, 0:1] + degp_ref[1][:, 0:1] + 2.0
    dis = lax.rsqrt(deg)
    for g in range(_NG):
        part = (dis * aggs[g][...] + (2.0 * dis) * us[g][...]
                + b_ref[0:1, g * _L:(g + 1) * _L])
        hbuf[:, g * _L:(g + 1) * _L] = jnp.maximum(part, 0.0)
    t = jnp.dot(hbuf[...], w_ref[...], preferred_element_type=jnp.float32)
    un = dis * t
    for g in range(_NG):
        outs[g][...] = un[:, g * _L:(g + 1) * _L]


_gcn_mid = pl.pallas_call(
    _gcn_mid_body,
    grid=(_GRID,),
    in_specs=[
        pl.BlockSpec((2, _BN, _L), lambda i: (0, i, 0)),
        pl.BlockSpec((_DP, _DP), lambda i: (0, 0)),
        pl.BlockSpec((1, _DP), lambda i: (0, 0)),
    ] + [pl.BlockSpec((_BN, _L), lambda i: (i, 0))] * (2 * _NG),
    out_specs=[pl.BlockSpec((_BN, _L), lambda i: (i, 0))] * _NG,
    out_shape=[jax.ShapeDtypeStruct((_N, _L), jnp.float32)] * _NG,
    scratch_shapes=[pltpu.VMEM((_BN, _DP), jnp.float32)],
)


def _h5_body(degp_ref, b_ref, *refs):
    aggs = refs[0:_NG]
    us = refs[_NG:2 * _NG]
    out = refs[2 * _NG]
    deg = degp_ref[0][:, 0:1] + degp_ref[1][:, 0:1] + 2.0
    dis = lax.rsqrt(deg)
    for g in range(_NG):
        part = (dis * aggs[g][...] + (2.0 * dis) * us[g][...]
                + b_ref[0:1, g * _L:(g + 1) * _L])
        out[:, g * _L:(g + 1) * _L] = jnp.maximum(part, 0.0)


_h5 = pl.pallas_call(
    _h5_body,
    grid=(_GRID,),
    in_specs=[
        pl.BlockSpec((2, _BN, _L), lambda i: (0, i, 0)),
        pl.BlockSpec((1, _DP), lambda i: (0, 0)),
    ] + [pl.BlockSpec((_BN, _L), lambda i: (i, 0))] * (2 * _NG),
    out_specs=pl.BlockSpec((_BN, _DP), lambda i: (i, 0)),
    out_shape=jax.ShapeDtypeStruct((_N, _DP), jnp.float32),
)


def _mlp_body(sums_ref, counts_ref, w1, bb1, w2, bb2, w3, bb3, out_ref):
    s = sums_ref[0] + sums_ref[1]
    c = counts_ref[0][:, 0:1] + counts_ref[1][:, 0:1]
    pooled = s / jnp.maximum(c, 1.0)
    h = jnp.maximum(
        jnp.dot(pooled, w1[...], preferred_element_type=jnp.float32) + bb1[...],
        0.0)
    h = jnp.maximum(
        jnp.dot(h, w2[...], preferred_element_type=jnp.float32) + bb2[...],
        0.0)
    o = jnp.dot(h, w3[...], preferred_element_type=jnp.float32) + bb3[...]
    out_ref[...] = o[:, :_DOUT]


_mlp = pl.pallas_call(
    _mlp_body,
    out_shape=jax.ShapeDtypeStruct((_G, _DOUT), jnp.float32),
)


def kernel(x, edge_index, batch, W1, b1, W2, b2, W3, b3, W4, b4, W5, b5,
           LW1, LB1, LW2, LB2, LW3, LB3):
    src2d = edge_index[0].reshape(_ER, _RW)
    dst2d = edge_index[1].reshape(_ER, _RW)
    batch2d = batch.reshape(_BR, _RW)

    pad = _DP - _DH
    W1p = jnp.pad(W1, ((0, 0), (0, pad)))
    Wps = [jnp.pad(Wi, ((0, pad), (0, pad))) for Wi in (W2, W3, W4, W5)]
    bps = [jnp.pad(bi, (0, pad)).reshape(1, _DP) for bi in (b1, b2, b3, b4, b5)]
    LW1p = jnp.pad(LW1, ((0, pad), (0, pad)))
    LW2p = jnp.pad(LW2, ((0, pad), (0, pad)))
    LW3p = jnp.pad(LW3, ((0, pad), (0, 3)))
    LB1p = jnp.pad(LB1, (0, pad)).reshape(1, _DP)
    LB2p = jnp.pad(LB2, (0, pad)).reshape(1, _DP)
    LB3p = jnp.pad(LB3, (0, 3)).reshape(1, _DOUT + 3)

    degp = _deg(dst2d)
    u = _k1(x, W1p, degp)
    for layer in range(4):
        agg = _edge_agg(src2d, dst2d, *u)
        u = _gcn_mid(degp, Wps[layer], bps[layer], *agg, *u)
    agg = _edge_agg(src2d, dst2d, *u)
    h5 = _h5(degp, bps[4], *agg, *u)
    sums_p, counts_p = _pool(h5.reshape(_BR, _RW, _DP), batch2d)
    return _mlp(sums_p, counts_p, LW1p, LB1p, LW2p, LB2p, LW3p, LB3p)


# speedup vs baseline: 7.2382x; 1.0057x over previous
import jax
import jax.numpy as jnp
from jax import lax
from jax.experimental import pallas as pl
from jax.experimental.pallas import tpu as pltpu
from jax.experimental.pallas import tpu_sc as plsc

_N = 50000
_E = 800000
_G = 64
_DIN = 336
_DH = 100
_DP = 112
_DOUT = 29
_NG = 7
_L = 16
_ER = 6400
_RW = 125
_TROWS = _ER // 16
_CH = 8
_NCH = _TROWS // _CH
_ZR = _N // 16
_ZC = _ZR // _RW
_OWNER = (0, 0, 0, 0, 1, 1, 1)
_BN = 2000
_GRID = _N // _BN
_DEG_TR = _ER // 32
_DEG_NCH = _DEG_TR // _CH
_BR = 400
_WCH = 1000
_NWC = _N // _WCH


def _acc_chunks(body):
    sub = lax.axis_index("s")

    def _k(k, c):
        ch = sub + 16 * k

        @pl.when(ch < _NWC)
        def _():
            body(ch)
        return c
    lax.fori_loop(0, (_NWC + 15) // 16, _k, 0)


def _sc_mesh():
    return plsc.VectorSubcoreMesh(core_axis_name="c", subcore_axis_name="s")


def _deg_body(dst_hbm, degp, didx, ones, zbuf, obuf, acc, ssem):
    core = lax.axis_index("c")
    sub = lax.axis_index("s")

    def _init(i, c):
        ones[i, :] = jnp.ones((_L,), jnp.float32)
        return c
    lax.fori_loop(0, _RW, _init, 0)

    def _initz(i, c):
        zbuf[i, :] = jnp.zeros((_L,), jnp.float32)
        return c
    lax.fori_loop(0, _WCH, _initz, 0)

    _acc_chunks(lambda ch: pltpu.sync_copy(zbuf, acc.at[pl.ds(ch * _WCH, _WCH)]))
    plsc.subcore_barrier()

    def _chunk(cidx, c):
        base = core * (_ER // 2) + sub * _DEG_TR + cidx * _CH
        pltpu.sync_copy(dst_hbm.at[pl.ds(base, _CH)], didx)
        cps = [pltpu.async_copy(ones, acc.at[didx.at[j]], ssem, add=True)
               for j in range(_CH)]
        for cp in cps:
            cp.wait()
        return c
    lax.fori_loop(0, _DEG_NCH, _chunk, 0)
    plsc.subcore_barrier()

    def _wout(ch):
        pltpu.sync_copy(acc.at[pl.ds(ch * _WCH, _WCH)], obuf)
        pltpu.sync_copy(obuf, degp.at[core, pl.ds(ch * _WCH, _WCH)])
    _acc_chunks(_wout)


_deg = pl.kernel(
    _deg_body,
    out_type=jax.ShapeDtypeStruct((2, _N, _L), jnp.float32),
    mesh=_sc_mesh(),
    compiler_params=pltpu.CompilerParams(use_tc_tiling_on_sc=False),
    scratch_types=[
        pltpu.VMEM((_CH, _RW), jnp.int32),
        pltpu.VMEM((_RW, _L), jnp.float32),
        pltpu.VMEM((_WCH, _L), jnp.float32),
        pltpu.VMEM((_WCH, _L), jnp.float32),
        pltpu.VMEM_SHARED((_N, _L), jnp.float32),
        pltpu.SemaphoreType.DMA,
    ],
)


def _edge_agg_body(src_hbm, dst_hbm, *refs):
    u = refs[0:_NG]
    agg = refs[_NG:2 * _NG]
    sidx, didx, rows, zbuf, obuf, acc, gsem, ssem = refs[2 * _NG:]
    core = lax.axis_index("c")
    sub = lax.axis_index("s")

    def _zb(i, c):
        zbuf[i, :] = jnp.zeros((_L,), jnp.float32)
        return c
    lax.fori_loop(0, _WCH, _zb, 0)

    for g in range(_NG):
        @pl.when(core == _OWNER[g])
        def _(g=g):
            _acc_chunks(
                lambda ch: pltpu.sync_copy(zbuf, acc.at[pl.ds(ch * _WCH, _WCH)]))
            plsc.subcore_barrier()

            def _chunk(cidx, c):
                base = sub * _TROWS + cidx * _CH
                pltpu.sync_copy(src_hbm.at[pl.ds(base, _CH)], sidx)
                pltpu.sync_copy(dst_hbm.at[pl.ds(base, _CH)], didx)
                gcps = [pltpu.async_copy(u[g].at[sidx.at[j]], rows.at[j], gsem)
                        for j in range(_CH)]
                for cp in gcps:
                    cp.wait()
                scps = [pltpu.async_copy(rows.at[j], acc.at[didx.at[j]], ssem,
                                         add=True)
                        for j in range(_CH)]
                for cp in scps:
                    cp.wait()
                return c
            lax.fori_loop(0, _NCH, _chunk, 0)
            plsc.subcore_barrier()

            def _wout(ch):
                pltpu.sync_copy(acc.at[pl.ds(ch * _WCH, _WCH)], obuf)
                pltpu.sync_copy(obuf, agg[g].at[pl.ds(ch * _WCH, _WCH)])
            _acc_chunks(_wout)
            plsc.subcore_barrier()


_edge_agg = pl.kernel(
    _edge_agg_body,
    out_type=[jax.ShapeDtypeStruct((_N, _L), jnp.float32)] * _NG,
    mesh=_sc_mesh(),
    compiler_params=pltpu.CompilerParams(use_tc_tiling_on_sc=False),
    scratch_types=[
        pltpu.VMEM((_CH, _RW), jnp.int32),
        pltpu.VMEM((_CH, _RW), jnp.int32),
        pltpu.VMEM((_CH, _RW, _L), jnp.float32),
        pltpu.VMEM((_WCH, _L), jnp.float32),
        pltpu.VMEM((_WCH, _L), jnp.float32),
        pltpu.VMEM_SHARED((_N, _L), jnp.float32),
        pltpu.SemaphoreType.DMA,
        pltpu.SemaphoreType.DMA,
    ],
)


def _pool_body(h5_hbm, b_hbm, sums_p, counts_p,
               bidx, hbuf, ones, zbuf, zcbuf, accs, accc, ssem):
    core = lax.axis_index("c")
    sub = lax.axis_index("s")
    w = core * 16 + sub

    def _init(i, c):
        ones[i, :] = jnp.ones((_L,), jnp.float32)
        return c
    lax.fori_loop(0, _RW, _init, 0)

    @pl.when(sub == 0)
    def _():
        def _z(i, c):
            for j in range(_DP // _L):
                zbuf[i, pl.ds(j * _L, _L)] = jnp.zeros((_L,), jnp.float32)
            zcbuf[i, :] = jnp.zeros((_L,), jnp.float32)
            return c
        lax.fori_loop(0, _G, _z, 0)
        pltpu.sync_copy(zbuf, accs)
        pltpu.sync_copy(zcbuf, accc)
    plsc.subcore_barrier()

    def _do_chunk(ch):
        pltpu.sync_copy(b_hbm.at[pl.ds(ch * _CH, _CH)], bidx)
        for j in range(_CH):
            row = ch * _CH + j
            pltpu.sync_copy(h5_hbm.at[row], hbuf)
            pltpu.sync_copy(hbuf, accs.at[bidx.at[j]], add=True)
            pltpu.sync_copy(ones, accc.at[bidx.at[j]], add=True)

    first = jnp.where(w < 18, 2 * w, w + 18)
    _do_chunk(first)

    @pl.when(w < 18)
    def _():
        _do_chunk(2 * w + 1)
    plsc.subcore_barrier()

    @pl.when(sub == 0)
    def _():
        pltpu.sync_copy(accs, zbuf)
        pltpu.sync_copy(zbuf, sums_p.at[core])
        pltpu.sync_copy(accc, zcbuf)
        pltpu.sync_copy(zcbuf, counts_p.at[core])


_pool = pl.kernel(
    _pool_body,
    out_type=[jax.ShapeDtypeStruct((2, _G, _DP), jnp.float32),
              jax.ShapeDtypeStruct((2, _G, _L), jnp.float32)],
    mesh=_sc_mesh(),
    compiler_params=pltpu.CompilerParams(use_tc_tiling_on_sc=False),
    scratch_types=[
        pltpu.VMEM((_CH, _RW), jnp.int32),
        pltpu.VMEM((_RW, _DP), jnp.float32),
        pltpu.VMEM((_RW, _L), jnp.float32),
        pltpu.VMEM((_G, _DP), jnp.float32),
        pltpu.VMEM((_G, _L), jnp.float32),
        pltpu.VMEM_SHARED((_G, _DP), jnp.float32),
        pltpu.VMEM_SHARED((_G, _L), jnp.float32),
        pltpu.SemaphoreType.DMA,
    ],
)


def _mm1_body(x_ref, w_ref, degp_ref, *outs):
    deg = degp_ref[0][:, 0:1] + degp_ref[1][:, 0:1] + 2.0
    dis = lax.rsqrt(deg)
    t = jnp.dot(x_ref[...], w_ref[...], preferred_element_type=jnp.float32)
    un = dis * t
    for g in range(_NG):
        outs[g][...] = un[:, g * _L:(g + 1) * _L]


_k1 = pl.pallas_call(
    _mm1_body,
    grid=(_GRID,),
    in_specs=[
        pl.BlockSpec((_BN, _DIN), lambda i: (i, 0)),
        pl.BlockSpec((_DIN, _DP), lambda i: (0, 0)),
        pl.BlockSpec((2, _BN, _L), lambda i: (0, i, 0)),
    ],
    out_specs=[pl.BlockSpec((_BN, _L), lambda i: (i, 0))] * _NG,
    out_shape=[jax.ShapeDtypeStruct((_N, _L), jnp.float32)] * _NG,
)


def _gcn_mid_body(degp_ref, w_ref, b_ref, *refs):
    aggs = refs[0:_NG]
    us = refs[_NG:2 * _NG]
    outs = refs[2 * _NG:3 * _NG]
    hbuf = refs[3 * _NG]
    deg = degp_ref[0][:, 0:1] + degp_ref[1][:, 0:1] + 2.0
    dis = lax.rsqrt(deg)
    for g in range(_NG):
        part = (dis * aggs[g][...] + (2.0 * dis) * us[g][...]
                + b_ref[0:1, g * _L:(g + 1) * _L])
        hbuf[:, g * _L:(g + 1) * _L] = jnp.maximum(part, 0.0)
    t = jnp.dot(hbuf[...], w_ref[...], preferred_element_type=jnp.float32)
    un = dis * t
    for g in range(_NG):
        outs[g][...] = un[:, g * _L:(g + 1) * _L]


_gcn_mid = pl.pallas_call(
    _gcn_mid_body,
    grid=(_GRID,),
    in_specs=[
        pl.BlockSpec((2, _BN, _L), lambda i: (0, i, 0)),
        pl.BlockSpec((_DP, _DP), lambda i: (0, 0)),
        pl.BlockSpec((1, _DP), lambda i: (0, 0)),
    ] + [pl.BlockSpec((_BN, _L), lambda i: (i, 0))] * (2 * _NG),
    out_specs=[pl.BlockSpec((_BN, _L), lambda i: (i, 0))] * _NG,
    out_shape=[jax.ShapeDtypeStruct((_N, _L), jnp.float32)] * _NG,
    scratch_shapes=[pltpu.VMEM((_BN, _DP), jnp.float32)],
)


def _h5_body(degp_ref, b_ref, *refs):
    aggs = refs[0:_NG]
    us = refs[_NG:2 * _NG]
    out = refs[2 * _NG]
    deg = degp_ref[0][:, 0:1] + degp_ref[1][:, 0:1] + 2.0
    dis = lax.rsqrt(deg)
    for g in range(_NG):
        part = (dis * aggs[g][...] + (2.0 * dis) * us[g][...]
                + b_ref[0:1, g * _L:(g + 1) * _L])
        out[:, g * _L:(g + 1) * _L] = jnp.maximum(part, 0.0)


_h5 = pl.pallas_call(
    _h5_body,
    grid=(_GRID,),
    in_specs=[
        pl.BlockSpec((2, _BN, _L), lambda i: (0, i, 0)),
        pl.BlockSpec((1, _DP), lambda i: (0, 0)),
    ] + [pl.BlockSpec((_BN, _L), lambda i: (i, 0))] * (2 * _NG),
    out_specs=pl.BlockSpec((_BN, _DP), lambda i: (i, 0)),
    out_shape=jax.ShapeDtypeStruct((_N, _DP), jnp.float32),
)


def _mlp_body(sums_ref, counts_ref, w1, bb1, w2, bb2, w3, bb3, out_ref):
    s = sums_ref[0] + sums_ref[1]
    c = counts_ref[0][:, 0:1] + counts_ref[1][:, 0:1]
    pooled = s / jnp.maximum(c, 1.0)
    h = jnp.maximum(
        jnp.dot(pooled, w1[...], preferred_element_type=jnp.float32) + bb1[...],
        0.0)
    h = jnp.maximum(
        jnp.dot(h, w2[...], preferred_element_type=jnp.float32) + bb2[...],
        0.0)
    o = jnp.dot(h, w3[...], preferred_element_type=jnp.float32) + bb3[...]
    out_ref[...] = o[:, :_DOUT]


_mlp = pl.pallas_call(
    _mlp_body,
    out_shape=jax.ShapeDtypeStruct((_G, _DOUT), jnp.float32),
)


def kernel(x, edge_index, batch, W1, b1, W2, b2, W3, b3, W4, b4, W5, b5,
           LW1, LB1, LW2, LB2, LW3, LB3):
    src2d = edge_index[0].reshape(_ER, _RW)
    dst2d = edge_index[1].reshape(_ER, _RW)
    batch2d = batch.reshape(_BR, _RW)

    pad = _DP - _DH
    W1p = jnp.pad(W1, ((0, 0), (0, pad)))
    Wps = [jnp.pad(Wi, ((0, pad), (0, pad))) for Wi in (W2, W3, W4, W5)]
    bps = [jnp.pad(bi, (0, pad)).reshape(1, _DP) for bi in (b1, b2, b3, b4, b5)]
    LW1p = jnp.pad(LW1, ((0, pad), (0, pad)))
    LW2p = jnp.pad(LW2, ((0, pad), (0, pad)))
    LW3p = jnp.pad(LW3, ((0, pad), (0, 3)))
    LB1p = jnp.pad(LB1, (0, pad)).reshape(1, _DP)
    LB2p = jnp.pad(LB2, (0, pad)).reshape(1, _DP)
    LB3p = jnp.pad(LB3, (0, 3)).reshape(1, _DOUT + 3)

    degp = _deg(dst2d)
    u = _k1(x, W1p, degp)
    for layer in range(4):
        agg = _edge_agg(src2d, dst2d, *u)
        u = _gcn_mid(degp, Wps[layer], bps[layer], *agg, *u)
    agg = _edge_agg(src2d, dst2d, *u)
    h5 = _h5(degp, bps[4], *agg, *u)
    sums_p, counts_p = _pool(h5.reshape(_BR, _RW, _DP), batch2d)
    return _mlp(sums_p, counts_p, LW1p, LB1p, LW2p, LB2p, LW3p, LB3p)


# speedup vs baseline: 10.5230x; 1.4538x over previous
import jax
import jax.numpy as jnp
from jax import lax
from jax.experimental import pallas as pl
from jax.experimental.pallas import tpu as pltpu
from jax.experimental.pallas import tpu_sc as plsc

_N = 50000
_E = 800000
_G = 64
_DIN = 336
_DH = 100
_DP = 112
_DOUT = 29
_NG = 7
_L = 16
_ER = 6400
_RW = 125
_TROWS = _ER // 16
_CH = 8
_NCH = _TROWS // _CH
_ZR = _N // 16
_ZC = _ZR // _RW
_OWNER = (0, 0, 0, 0, 1, 1, 1)
_BN = 2000
_GRID = _N // _BN
_DEG_TR = _ER // 32
_DEG_NCH = _DEG_TR // _CH
_BR = 400
_WCH = 500
_NWC = _N // _WCH


def _acc_chunks(body):
    sub = lax.axis_index("s")

    def _k(k, c):
        ch = sub + 16 * k

        @pl.when(ch < _NWC)
        def _():
            body(ch)
        return c
    lax.fori_loop(0, (_NWC + 15) // 16, _k, 0)


def _sc_mesh():
    return plsc.VectorSubcoreMesh(core_axis_name="c", subcore_axis_name="s")


def _deg_body(dst_hbm, degp, didx, ones, zbuf, obuf, acc, ssem):
    core = lax.axis_index("c")
    sub = lax.axis_index("s")

    def _init(i, c):
        ones[i, :] = jnp.ones((_L,), jnp.float32)
        return c
    lax.fori_loop(0, _RW, _init, 0)

    def _initz(i, c):
        zbuf[i, :] = jnp.zeros((_L,), jnp.float32)
        return c
    lax.fori_loop(0, _WCH, _initz, 0)

    _acc_chunks(lambda ch: pltpu.sync_copy(zbuf, acc.at[pl.ds(ch * _WCH, _WCH)]))
    plsc.subcore_barrier()

    def _chunk(cidx, c):
        base = core * (_ER // 2) + sub * _DEG_TR + cidx * _CH
        pltpu.sync_copy(dst_hbm.at[pl.ds(base, _CH)], didx)
        cps = [pltpu.async_copy(ones, acc.at[didx.at[j]], ssem, add=True)
               for j in range(_CH)]
        for cp in cps:
            cp.wait()
        return c
    lax.fori_loop(0, _DEG_NCH, _chunk, 0)
    plsc.subcore_barrier()

    def _wout(ch):
        pltpu.sync_copy(acc.at[pl.ds(ch * _WCH, _WCH)], obuf)
        pltpu.sync_copy(obuf, degp.at[core, pl.ds(ch * _WCH, _WCH)])
    _acc_chunks(_wout)


_deg = pl.kernel(
    _deg_body,
    out_type=jax.ShapeDtypeStruct((2, _N, _L), jnp.float32),
    mesh=_sc_mesh(),
    compiler_params=pltpu.CompilerParams(use_tc_tiling_on_sc=False),
    scratch_types=[
        pltpu.VMEM((_CH, _RW), jnp.int32),
        pltpu.VMEM((_RW, _L), jnp.float32),
        pltpu.VMEM((_WCH, _L), jnp.float32),
        pltpu.VMEM((_WCH, _L), jnp.float32),
        pltpu.VMEM_SHARED((_N, _L), jnp.float32),
        pltpu.SemaphoreType.DMA,
    ],
)


_CHR = 10


def _edge_agg_body(src_hbm, dst_hbm, *refs):
    u = refs[0:_NG]
    agg = refs[_NG:2 * _NG + 1]
    sidx, didx, rows, zbuf, obuf, acc, gsem, ssem, isem = refs[2 * _NG + 1:]
    core = lax.axis_index("c")
    sub = lax.axis_index("s")

    def _zb(i, c):
        zbuf[i, :] = jnp.zeros((_L,), jnp.float32)
        return c
    lax.fori_loop(0, _WCH, _zb, 0)

    def _process(u_ref, out_ref, row_base, rows_tile):
        nc = rows_tile // _CHR

        def _ibase(c):
            return row_base + sub * rows_tile + c * _CHR

        def _issue_idx(c, sync=False):
            s = lax.rem(c, 4)
            if sync:
                pltpu.sync_copy(src_hbm.at[pl.ds(_ibase(c), _CHR)], sidx.at[s])
                pltpu.sync_copy(dst_hbm.at[pl.ds(_ibase(c), _CHR)], didx.at[s])
            else:
                pltpu.async_copy(src_hbm.at[pl.ds(_ibase(c), _CHR)],
                                 sidx.at[s], isem)
                pltpu.async_copy(dst_hbm.at[pl.ds(_ibase(c), _CHR)],
                                 didx.at[s], isem)

        def _wait_idx(c):
            s = lax.rem(c, 4)
            pltpu.make_async_copy(src_hbm.at[pl.ds(_ibase(c), _CHR)],
                                  sidx.at[s], isem).wait()
            pltpu.make_async_copy(dst_hbm.at[pl.ds(_ibase(c), _CHR)],
                                  didx.at[s], isem).wait()

        def _issue_gath(c):
            s = lax.rem(c, 4)
            rb = lax.rem(c, 2)
            for j in range(_CHR):
                pltpu.async_copy(u_ref.at[sidx.at[s, j]], rows.at[rb, j], gsem)

        def _wait_gath(c):
            s = lax.rem(c, 4)
            rb = lax.rem(c, 2)
            for j in range(_CHR):
                pltpu.make_async_copy(u_ref.at[sidx.at[s, j]],
                                      rows.at[rb, j], gsem).wait()

        def _issue_scat(c):
            s = lax.rem(c, 4)
            rb = lax.rem(c, 2)
            for j in range(_CHR):
                pltpu.async_copy(rows.at[rb, j], acc.at[didx.at[s, j]], ssem,
                                 add=True)

        def _wait_scat(c):
            s = lax.rem(c, 4)
            rb = lax.rem(c, 2)
            for j in range(_CHR):
                pltpu.make_async_copy(rows.at[rb, j],
                                      acc.at[didx.at[s, j]], ssem).wait()

        _issue_idx(0, sync=True)
        _issue_idx(1)
        _issue_gath(0)

        def _step(c, carry):
            _wait_gath(c)

            @pl.when(c > 0)
            def _():
                _wait_scat(c - 1)
            _issue_scat(c)

            @pl.when(c + 1 < nc)
            def _():
                _wait_idx(c + 1)
                _issue_gath(c + 1)

                @pl.when(c + 2 < nc)
                def _():
                    _issue_idx(c + 2)
            return carry
        lax.fori_loop(0, nc, _step, 0)
        _wait_scat(nc - 1)
        plsc.subcore_barrier()

        def _wout(ch):
            pltpu.sync_copy(acc.at[pl.ds(ch * _WCH, _WCH)], obuf)
            pltpu.sync_copy(obuf, out_ref.at[pl.ds(ch * _WCH, _WCH)])
        _acc_chunks(_wout)
        plsc.subcore_barrier()

    def _task(u_ref, out_ref, row_base, rows_tile):
        _acc_chunks(
            lambda ch: pltpu.sync_copy(zbuf, acc.at[pl.ds(ch * _WCH, _WCH)]))
        plsc.subcore_barrier()
        _process(u_ref, out_ref, row_base, rows_tile)

    @pl.when(core == 0)
    def _():
        for g in (0, 1, 2):
            _task(u[g], agg[g], 0, _TROWS)
        _task(u[3], agg[3], 0, _TROWS // 2)

    @pl.when(core == 1)
    def _():
        for g in (4, 5, 6):
            _task(u[g], agg[g], 0, _TROWS)
        _task(u[3], agg[7], _ER // 2, _TROWS // 2)


_edge_agg = pl.kernel(
    _edge_agg_body,
    out_type=[jax.ShapeDtypeStruct((_N, _L), jnp.float32)] * (_NG + 1),
    mesh=_sc_mesh(),
    compiler_params=pltpu.CompilerParams(use_tc_tiling_on_sc=False),
    scratch_types=[
        pltpu.VMEM((4, _CHR, _RW), jnp.int32),
        pltpu.VMEM((4, _CHR, _RW), jnp.int32),
        pltpu.VMEM((2, _CHR, _RW, _L), jnp.float32),
        pltpu.VMEM((_WCH, _L), jnp.float32),
        pltpu.VMEM((_WCH, _L), jnp.float32),
        pltpu.VMEM_SHARED((_N, _L), jnp.float32),
        pltpu.SemaphoreType.DMA,
        pltpu.SemaphoreType.DMA,
        pltpu.SemaphoreType.DMA,
    ],
)


def _pool_body(h5_hbm, b_hbm, sums_p, counts_p,
               bidx, hbuf, ones, zbuf, zcbuf, accs, accc, ssem):
    core = lax.axis_index("c")
    sub = lax.axis_index("s")
    w = core * 16 + sub

    def _init(i, c):
        ones[i, :] = jnp.ones((_L,), jnp.float32)
        return c
    lax.fori_loop(0, _RW, _init, 0)

    @pl.when(sub == 0)
    def _():
        def _z(i, c):
            for j in range(_DP // _L):
                zbuf[i, pl.ds(j * _L, _L)] = jnp.zeros((_L,), jnp.float32)
            zcbuf[i, :] = jnp.zeros((_L,), jnp.float32)
            return c
        lax.fori_loop(0, _G, _z, 0)
        pltpu.sync_copy(zbuf, accs)
        pltpu.sync_copy(zcbuf, accc)
    plsc.subcore_barrier()

    def _do_chunk(ch):
        pltpu.sync_copy(b_hbm.at[pl.ds(ch * _CH, _CH)], bidx)
        for j in range(_CH):
            row = ch * _CH + j
            pltpu.sync_copy(h5_hbm.at[row], hbuf)
            pltpu.sync_copy(hbuf, accs.at[bidx.at[j]], add=True)
            pltpu.sync_copy(ones, accc.at[bidx.at[j]], add=True)

    first = jnp.where(w < 18, 2 * w, w + 18)
    _do_chunk(first)

    @pl.when(w < 18)
    def _():
        _do_chunk(2 * w + 1)
    plsc.subcore_barrier()

    @pl.when(sub == 0)
    def _():
        pltpu.sync_copy(accs, zbuf)
        pltpu.sync_copy(zbuf, sums_p.at[core])
        pltpu.sync_copy(accc, zcbuf)
        pltpu.sync_copy(zcbuf, counts_p.at[core])


_pool = pl.kernel(
    _pool_body,
    out_type=[jax.ShapeDtypeStruct((2, _G, _DP), jnp.float32),
              jax.ShapeDtypeStruct((2, _G, _L), jnp.float32)],
    mesh=_sc_mesh(),
    compiler_params=pltpu.CompilerParams(use_tc_tiling_on_sc=False),
    scratch_types=[
        pltpu.VMEM((_CH, _RW), jnp.int32),
        pltpu.VMEM((_RW, _DP), jnp.float32),
        pltpu.VMEM((_RW, _L), jnp.float32),
        pltpu.VMEM((_G, _DP), jnp.float32),
        pltpu.VMEM((_G, _L), jnp.float32),
        pltpu.VMEM_SHARED((_G, _DP), jnp.float32),
        pltpu.VMEM_SHARED((_G, _L), jnp.float32),
        pltpu.SemaphoreType.DMA,
    ],
)


def _mm1_body(x_ref, w_ref, degp_ref, *outs):
    deg = degp_ref[0][:, 0:1] + degp_ref[1][:, 0:1] + 2.0
    dis = lax.rsqrt(deg)
    t = jnp.dot(x_ref[...], w_ref[...], preferred_element_type=jnp.float32)
    un = dis * t
    for g in range(_NG):
        outs[g][...] = un[:, g * _L:(g + 1) * _L]


_k1 = pl.pallas_call(
    _mm1_body,
    grid=(_GRID,),
    in_specs=[
        pl.BlockSpec((_BN, _DIN), lambda i: (i, 0)),
        pl.BlockSpec((_DIN, _DP), lambda i: (0, 0)),
        pl.BlockSpec((2, _BN, _L), lambda i: (0, i, 0)),
    ],
    out_specs=[pl.BlockSpec((_BN, _L), lambda i: (i, 0))] * _NG,
    out_shape=[jax.ShapeDtypeStruct((_N, _L), jnp.float32)] * _NG,
)


def _gcn_mid_body(degp_ref, w_ref, b_ref, *refs):
    aggs = refs[0:_NG + 1]
    us = refs[_NG + 1:2 * _NG + 1]
    outs = refs[2 * _NG + 1:3 * _NG + 1]
    hbuf = refs[3 * _NG + 1]
    deg = degp_ref[0][:, 0:1] + degp_ref[1][:, 0:1] + 2.0
    dis = lax.rsqrt(deg)
    for g in range(_NG):
        a = aggs[g][...]
        if g == 3:
            a = a + aggs[_NG][...]
        part = (dis * a + (2.0 * dis) * us[g][...]
                + b_ref[0:1, g * _L:(g + 1) * _L])
        hbuf[:, g * _L:(g + 1) * _L] = jnp.maximum(part, 0.0)
    t = jnp.dot(hbuf[...], w_ref[...], preferred_element_type=jnp.float32)
    un = dis * t
    for g in range(_NG):
        outs[g][...] = un[:, g * _L:(g + 1) * _L]


_gcn_mid = pl.pallas_call(
    _gcn_mid_body,
    grid=(_GRID,),
    in_specs=[
        pl.BlockSpec((2, _BN, _L), lambda i: (0, i, 0)),
        pl.BlockSpec((_DP, _DP), lambda i: (0, 0)),
        pl.BlockSpec((1, _DP), lambda i: (0, 0)),
    ] + [pl.BlockSpec((_BN, _L), lambda i: (i, 0))] * (2 * _NG + 1),
    out_specs=[pl.BlockSpec((_BN, _L), lambda i: (i, 0))] * _NG,
    out_shape=[jax.ShapeDtypeStruct((_N, _L), jnp.float32)] * _NG,
    scratch_shapes=[pltpu.VMEM((_BN, _DP), jnp.float32)],
)


def _h5_body(degp_ref, b_ref, *refs):
    aggs = refs[0:_NG + 1]
    us = refs[_NG + 1:2 * _NG + 1]
    out = refs[2 * _NG + 1]
    deg = degp_ref[0][:, 0:1] + degp_ref[1][:, 0:1] + 2.0
    dis = lax.rsqrt(deg)
    for g in range(_NG):
        a = aggs[g][...]
        if g == 3:
            a = a + aggs[_NG][...]
        part = (dis * a + (2.0 * dis) * us[g][...]
                + b_ref[0:1, g * _L:(g + 1) * _L])
        out[:, g * _L:(g + 1) * _L] = jnp.maximum(part, 0.0)


_h5 = pl.pallas_call(
    _h5_body,
    grid=(_GRID,),
    in_specs=[
        pl.BlockSpec((2, _BN, _L), lambda i: (0, i, 0)),
        pl.BlockSpec((1, _DP), lambda i: (0, 0)),
    ] + [pl.BlockSpec((_BN, _L), lambda i: (i, 0))] * (2 * _NG + 1),
    out_specs=pl.BlockSpec((_BN, _DP), lambda i: (i, 0)),
    out_shape=jax.ShapeDtypeStruct((_N, _DP), jnp.float32),
)


def _mlp_body(sums_ref, counts_ref, w1, bb1, w2, bb2, w3, bb3, out_ref):
    s = sums_ref[0] + sums_ref[1]
    c = counts_ref[0][:, 0:1] + counts_ref[1][:, 0:1]
    pooled = s / jnp.maximum(c, 1.0)
    h = jnp.maximum(
        jnp.dot(pooled, w1[...], preferred_element_type=jnp.float32) + bb1[...],
        0.0)
    h = jnp.maximum(
        jnp.dot(h, w2[...], preferred_element_type=jnp.float32) + bb2[...],
        0.0)
    o = jnp.dot(h, w3[...], preferred_element_type=jnp.float32) + bb3[...]
    out_ref[...] = o[:, :_DOUT]


_mlp = pl.pallas_call(
    _mlp_body,
    out_shape=jax.ShapeDtypeStruct((_G, _DOUT), jnp.float32),
)


def kernel(x, edge_index, batch, W1, b1, W2, b2, W3, b3, W4, b4, W5, b5,
           LW1, LB1, LW2, LB2, LW3, LB3):
    src2d = edge_index[0].reshape(_ER, _RW)
    dst2d = edge_index[1].reshape(_ER, _RW)
    batch2d = batch.reshape(_BR, _RW)

    pad = _DP - _DH
    W1p = jnp.pad(W1, ((0, 0), (0, pad)))
    Wps = [jnp.pad(Wi, ((0, pad), (0, pad))) for Wi in (W2, W3, W4, W5)]
    bps = [jnp.pad(bi, (0, pad)).reshape(1, _DP) for bi in (b1, b2, b3, b4, b5)]
    LW1p = jnp.pad(LW1, ((0, pad), (0, pad)))
    LW2p = jnp.pad(LW2, ((0, pad), (0, pad)))
    LW3p = jnp.pad(LW3, ((0, pad), (0, 3)))
    LB1p = jnp.pad(LB1, (0, pad)).reshape(1, _DP)
    LB2p = jnp.pad(LB2, (0, pad)).reshape(1, _DP)
    LB3p = jnp.pad(LB3, (0, 3)).reshape(1, _DOUT + 3)

    degp = _deg(dst2d)
    u = _k1(x, W1p, degp)
    for layer in range(4):
        agg = _edge_agg(src2d, dst2d, *u)
        u = _gcn_mid(degp, Wps[layer], bps[layer], *agg, *u)
    agg = _edge_agg(src2d, dst2d, *u)
    h5 = _h5(degp, bps[4], *agg, *u)
    sums_p, counts_p = _pool(h5.reshape(_BR, _RW, _DP), batch2d)
    return _mlp(sums_p, counts_p, LW1p, LB1p, LW2p, LB2p, LW3p, LB3p)
